# baseline (speedup 1.0000x reference)
import functools
import jax
import jax.numpy as jnp
from jax import lax
from jax.experimental import pallas as pl
from jax.experimental.pallas import tpu as pltpu
from jax.experimental.pallas import tpu_sc as plsc

VOCAB = 100000
DIM = 64
PAD = 0

NC = 2
NS = 16
NW = NC * NS

BATCH = 1024
SEQ = 20
INNER = 20
NSI = SEQ * INNER
FPW = DIM // NW
L = 16
NR = 4
G = 2


def _emb_body(idxT_hbm, wT_hbm, out_hbm, wrow_v, idx_v, ob_v, wsem,
              isem, osem):
    wid = lax.axis_index("s") * NC + lax.axis_index("c")

    def idx_copy(si, r):
        return pltpu.make_async_copy(idxT_hbm.at[si], idx_v.at[r],
                                     isem.at[r])

    def out_copy(si, d, r):
        return pltpu.make_async_copy(ob_v.at[r], out_hbm.at[si, d],
                                     osem.at[r])

    for p in range(FPW):
        d = wid * FPW + p
        pltpu.make_async_copy(wT_hbm.at[d], wrow_v, wsem).start()
        pltpu.make_async_copy(wT_hbm.at[d], wrow_v, wsem).wait()
        head = wrow_v[pl.ds(0, L)]
        wrow_v[pl.ds(0, L)] = jnp.where(
            lax.iota(jnp.int32, L) == PAD, jnp.float32(0), head)

        for si in range(G):
            idx_copy(si, si).start()

        def si_round(t, carry):
            for r in range(NR):
                si = t * NR + r
                rp = (r + G) % NR

                @pl.when(si + G < NSI)
                def _():
                    idx_copy(si + G, rp).start()

                idx_copy(si, r).wait()

                @pl.when(si >= NR)
                def _():
                    out_copy(si - NR, d, r).wait()

                for g in range(BATCH // L):
                    iv = idx_v[r, pl.ds(g * L, L)]
                    ob_v[r, pl.ds(g * L, L)] = plsc.load_gather(
                        wrow_v, [iv])

                out_copy(si, d, r).start()
            return carry

        lax.fori_loop(0, NSI // NR, si_round, 0)

        for r in range(NR):
            out_copy(NSI - NR + r, d, r).wait()


@functools.partial(jax.jit, static_argnames=())
def _run(idxT, WT):
    mesh = plsc.VectorSubcoreMesh(core_axis_name="c", subcore_axis_name="s")
    f = pl.kernel(
        _emb_body,
        out_type=jax.ShapeDtypeStruct((NSI, DIM, BATCH), jnp.float32),
        mesh=mesh,
        scratch_types=[
            pltpu.VMEM((VOCAB,), jnp.float32),
            pltpu.VMEM((NR, BATCH), jnp.int32),
            pltpu.VMEM((NR, BATCH), jnp.float32),
            pltpu.SemaphoreType.DMA,
            pltpu.SemaphoreType.DMA((NR,)),
            pltpu.SemaphoreType.DMA((NR,)),
        ],
        compiler_params=pltpu.CompilerParams(
            needs_layout_passes=False, use_tc_tiling_on_sc=False),
    )
    return f(idxT, WT)


def kernel(token_ids, W):
    idxT = jnp.transpose(token_ids.astype(jnp.int32), (1, 2, 0))
    idxT = idxT.reshape(NSI, BATCH)
    WT = jnp.transpose(W, (1, 0))
    y = _run(idxT, WT)
    y = y.reshape(SEQ, INNER, DIM, BATCH)
    return jnp.transpose(y, (3, 0, 1, 2))


# speedup vs baseline: 1.8826x; 1.8826x over previous
import functools
import jax
import jax.numpy as jnp
from jax import lax
from jax.experimental import pallas as pl
from jax.experimental.pallas import tpu as pltpu
from jax.experimental.pallas import tpu_sc as plsc

VOCAB = 100000
DIM = 64
PAD = 0

NC = 2
NS = 16
NW = NC * NS

BATCH = 1024
SEQ = 20
INNER = 20
B = BATCH * SEQ * INNER
BPW = B // NW
UNIT = 128
NU = BPW // UNIT
NB = 10
G = 5

JROWS = B * DIM // 128
JPB = JROWS // BATCH


def _gather_body(idx_hbm, table_hbm, out2d, idx_v, rows_v, gsem, osem):
    wid = lax.axis_index("s") * NC + lax.axis_index("c")
    base = wid * BPW
    pltpu.sync_copy(idx_hbm.at[pl.ds(base, BPW)], idx_v)

    def gather_copy(u, b):
        return pltpu.make_async_copy(
            table_hbm.at[idx_v.at[pl.ds(u * UNIT, UNIT)]],
            rows_v.at[b], gsem.at[b])

    def out_copy(u, b):
        return pltpu.make_async_copy(
            rows_v.at[b], out2d.at[pl.ds(base + u * UNIT, UNIT)],
            osem.at[b])

    def fixup(u, b):
        def group_fix(g, c2):
            goff = u * UNIT + g * 16
            iv = idx_v[pl.ds(goff, 16)]
            has_pad = jnp.min(iv, axis=0) == PAD

            @pl.when(has_pad)
            def _():
                m = iv == PAD
                row_idx = g * 16 + lax.iota(jnp.int32, 16)
                zeros = jnp.zeros((16,), jnp.float32)
                for c in range(DIM):
                    col_idx = jnp.full((16,), c, jnp.int32)
                    plsc.store_scatter(rows_v.at[b], [row_idx, col_idx],
                                       zeros, mask=m)
            return c2
        lax.fori_loop(0, UNIT // 16, group_fix, 0)

    for u in range(G):
        gather_copy(u, u).start()

    def round_body(t, carry):
        for b in range(NB):
            u = t * NB + b
            up = u + G
            bp = (b + G) % NB

            @pl.when(jnp.logical_and(up < NU, up >= NB))
            def _():
                out_copy(up - NB, bp).wait()

            @pl.when(up < NU)
            def _():
                gather_copy(up, bp).start()

            gather_copy(u, b).wait()
            fixup(u, b)
            out_copy(u, b).start()
        return carry

    lax.fori_loop(0, NU // NB, round_body, 0)

    for b in range(NB):
        out_copy((NU // NB - 1) * NB + b, b).wait()


def _transpose_body(x_ref, y_ref):
    for k in range(8):
        y_ref[k] = x_ref[:, k, :].T


@functools.partial(jax.jit, static_argnames=())
def _run(idx_flat, W):
    mesh = plsc.VectorSubcoreMesh(core_axis_name="c", subcore_axis_name="s")
    gather = pl.kernel(
        _gather_body,
        out_type=jax.ShapeDtypeStruct((B, DIM), jnp.float32),
        mesh=mesh,
        scratch_types=[
            pltpu.VMEM((BPW,), jnp.int32),
            pltpu.VMEM((NB, UNIT, DIM), jnp.float32),
            pltpu.SemaphoreType.DMA((NB,)),
            pltpu.SemaphoreType.DMA((NB,)),
        ],
        compiler_params=pltpu.CompilerParams(
            needs_layout_passes=False, use_tc_tiling_on_sc=False),
    )
    x = gather(idx_flat, W).reshape(BATCH, JPB, 128)

    y = pl.pallas_call(
        _transpose_body,
        grid=(JPB // 8,),
        in_specs=[pl.BlockSpec((BATCH, 8, 128), lambda j: (0, j, 0))],
        out_specs=pl.BlockSpec((8, 128, BATCH), lambda j: (j, 0, 0)),
        out_shape=jax.ShapeDtypeStruct((JPB, 128, BATCH), jnp.float32),
    )(x)

    y = y.reshape(SEQ, INNER, DIM, BATCH)
    return jnp.transpose(y, (3, 0, 1, 2))


def kernel(token_ids, W):
    idx_flat = token_ids.reshape(-1).astype(jnp.int32)
    return _run(idx_flat, W)
